# baseline (speedup 1.0000x reference)
import functools

import jax
import jax.numpy as jnp
from jax import lax
from jax.experimental import pallas as pl
from jax.experimental.pallas import tpu as pltpu
from jax.experimental.pallas import tpu_sc as plsc

NC = 2
NS = 16
NW = NC * NS
CHUNK = 48
NBUF = 6


@functools.lru_cache(maxsize=None)
def _sc_aggregate(n: int, nf: int, d: int, e: int, c: int, with_deg: bool):
    epw = e // NW
    assert epw * NW == e and epw % c == 0
    kpw = epw // c
    assert kpw % 8 == 0 and kpw % NBUF == 0
    rpt = n // NS
    assert rpt * NS == n and rpt % 8 == 0
    kg = next(g for g in range(min(kpw, 40), 0, -8)
              if kpw % g == 0 and g % NBUF == 0)
    zc = next(z for z in range(min(c, 128) - min(c, 128) % 8, 0, -8)
              if rpt % z == 0)
    nlane = d // 16
    caux = -(-c // 16) * 16

    mesh = plsc.VectorSubcoreMesh(core_axis_name="c", subcore_axis_name="s")
    out_type = [jax.ShapeDtypeStruct((NC, n, d), jnp.float32)]
    if with_deg:
        out_type.append(jax.ShapeDtypeStruct((NC, n), jnp.float32))

    scratch = (
        [pltpu.VMEM((kg, c), jnp.int32)] * 2
        + [pltpu.VMEM((c, d), jnp.float32)] * NBUF
        + [pltpu.VMEM_SHARED((n, d), jnp.float32)]
        + [pltpu.SemaphoreType.DMA] * (2 * NBUF)
    )
    if with_deg:
        scratch += [
            pltpu.VMEM((caux,), jnp.float32),
            pltpu.VMEM_SHARED((n,), jnp.float32),
            pltpu.SemaphoreType.DMA,
        ]

    def body(feats_hbm, src_hbm, dst_hbm, *refs):
        if with_deg:
            (outp_hbm, outdeg_hbm, src_v, dst_v, *rest) = refs
            aux1d, deg_sh, dsem = rest[-3:]
            rest = rest[:-3]
        else:
            (outp_hbm, src_v, dst_v, *rest) = refs
        rows = tuple(rest[:NBUF])
        agg_sh = rest[NBUF]
        gsem = tuple(rest[NBUF + 1:NBUF + 1 + NBUF])
        ssem = tuple(rest[NBUF + 1 + NBUF:NBUF + 1 + 2 * NBUF])
        cid = lax.axis_index("c")
        sid = lax.axis_index("s")
        wid = cid * NS + sid

        zv = jnp.zeros((16,), jnp.float32)

        def zrow(r, _):
            for j in range(nlane):
                rows[0][r, pl.ds(j * 16, 16)] = zv
            return 0

        lax.fori_loop(0, c, zrow, 0)
        if with_deg:
            ov = jnp.ones((16,), jnp.float32)
            for j in range(caux // 16):
                aux1d[pl.ds(j * 16, 16)] = zv

        for t in range(rpt // zc):
            pltpu.sync_copy(rows[0].at[pl.ds(0, zc)],
                            agg_sh.at[pl.ds(sid * rpt + t * zc, zc)])
        if with_deg:
            for t in range(-(-rpt // caux)):
                w = min(caux, rpt - t * caux)
                pltpu.sync_copy(
                    aux1d.at[pl.ds(0, w)],
                    deg_sh.at[pl.ds(sid * rpt + t * caux, w)])
            for j in range(caux // 16):
                aux1d[pl.ds(j * 16, 16)] = ov
        plsc.subcore_barrier()

        def gather(k, b):
            return pltpu.async_copy(feats_hbm.at[src_v.at[k]], rows[b],
                                    gsem[b])

        def scatter(k, b):
            return pltpu.async_copy(rows[b], agg_sh.at[dst_v.at[k]],
                                    ssem[b], add=True)

        def wait_g(b):
            pltpu.make_async_copy(feats_hbm.at[src_v.at[0]], rows[b],
                                  gsem[b]).wait()

        def wait_s(b):
            pltpu.make_async_copy(rows[b], agg_sh.at[dst_v.at[0]],
                                  ssem[b]).wait()

        nsteps = kg // NBUF
        for g in range(kpw // kg):
            pltpu.sync_copy(src_hbm.at[pl.ds(wid * kpw + g * kg, kg)], src_v)
            pltpu.sync_copy(dst_hbm.at[pl.ds(wid * kpw + g * kg, kg)], dst_v)
            for b in range(NBUF):
                gather(b, b)

            def step(t, _):
                k0 = NBUF * t
                for b in range(NBUF):
                    wait_g(b)
                    scatter(k0 + b, b)
                    if with_deg:
                        pltpu.async_copy(aux1d.at[pl.ds(0, c)],
                                         deg_sh.at[dst_v.at[k0 + b]], dsem,
                                         add=True)

                @pl.when(t < nsteps - 1)
                def _():
                    for b in range(NBUF):
                        wait_s(b)
                        gather(k0 + NBUF + b, b)

                return 0

            lax.fori_loop(0, nsteps, step, 0)
            for b in range(NBUF):
                wait_s(b)
            if with_deg:
                def ddrain(t, _):
                    pltpu.make_async_copy(aux1d.at[pl.ds(0, c)],
                                          deg_sh.at[dst_v.at[0]],
                                          dsem).wait()
                    return 0

                lax.fori_loop(0, kg, ddrain, 0)

        plsc.subcore_barrier()

        pltpu.sync_copy(agg_sh.at[pl.ds(sid * rpt, rpt)],
                        outp_hbm.at[cid, pl.ds(sid * rpt, rpt)])
        if with_deg:
            pltpu.sync_copy(deg_sh.at[pl.ds(sid * rpt, rpt)],
                            outdeg_hbm.at[cid, pl.ds(sid * rpt, rpt)])

    return pl.kernel(body, out_type=out_type, mesh=mesh,
                     scratch_types=scratch)


def _tc_pre(xin, Wr, bl, block: int):
    n, d = xin.shape

    def body(x_ref, wr_ref, bl_ref, o_ref):
        o_ref[...] = (jnp.dot(x_ref[...], wr_ref[...],
                              preferred_element_type=jnp.float32)
                      + bl_ref[...])

    return pl.pallas_call(
        body,
        grid=(n // block,),
        in_specs=[
            pl.BlockSpec((block, d), lambda i: (i, 0)),
            pl.BlockSpec((d, d), lambda i: (0, 0)),
            pl.BlockSpec((1, d), lambda i: (0, 0)),
        ],
        out_specs=pl.BlockSpec((block, d), lambda i: (i, 0)),
        out_shape=jax.ShapeDtypeStruct((n, d), jnp.float32),
    )(xin, Wr, bl)


def _tc_post1(p, degp, pre, Wl, block: int):
    n, d = pre.shape

    def body(p_ref, deg_ref, pre_ref, wl_ref, o_ref, inv_ref):
        dsum = deg_ref[0] + deg_ref[1]
        invd = 1.0 / jnp.maximum(dsum, 1.0)
        agg = (p_ref[0] + p_ref[1]) * invd
        y = (jnp.dot(agg, wl_ref[...], preferred_element_type=jnp.float32)
             + pre_ref[...])
        o_ref[...] = jnp.maximum(y, 0.0)
        inv_ref[...] = invd

    return pl.pallas_call(
        body,
        grid=(n // block,),
        in_specs=[
            pl.BlockSpec((2, block, d), lambda i: (0, i, 0)),
            pl.BlockSpec((2, block, 1), lambda i: (0, i, 0)),
            pl.BlockSpec((block, d), lambda i: (i, 0)),
            pl.BlockSpec((d, d), lambda i: (0, 0)),
        ],
        out_specs=[
            pl.BlockSpec((block, d), lambda i: (i, 0)),
            pl.BlockSpec((block, 1), lambda i: (i, 0)),
        ],
        out_shape=[
            jax.ShapeDtypeStruct((n, d), jnp.float32),
            jax.ShapeDtypeStruct((n, 1), jnp.float32),
        ],
    )(p, degp, pre, Wl)


def _tc_post2(p, invd, pre, Wl, block: int):
    n, d = pre.shape

    def body(p_ref, inv_ref, pre_ref, wl_ref, o_ref):
        agg = (p_ref[0] + p_ref[1]) * inv_ref[...]
        o_ref[...] = (
            jnp.dot(agg, wl_ref[...], preferred_element_type=jnp.float32)
            + pre_ref[...])

    return pl.pallas_call(
        body,
        grid=(n // block,),
        in_specs=[
            pl.BlockSpec((2, block, d), lambda i: (0, i, 0)),
            pl.BlockSpec((block, 1), lambda i: (i, 0)),
            pl.BlockSpec((block, d), lambda i: (i, 0)),
            pl.BlockSpec((d, d), lambda i: (0, 0)),
        ],
        out_specs=pl.BlockSpec((block, d), lambda i: (i, 0)),
        out_shape=jax.ShapeDtypeStruct((n, d), jnp.float32),
    )(p, invd, pre, Wl)


def kernel(x, edge_index, Wl1, bl1, Wr1, Wl2, bl2, Wr2):
    n, d = x.shape
    e = edge_index.shape[1]
    ei = edge_index.astype(jnp.int32)

    align = NS * 128
    n2 = -(-n // align) * align
    block = next(b for b in range(1024, 0, -8) if n % b == 0)

    c = CHUNK
    egrain = NW * c * 8
    e2 = -(-e // egrain) * egrain
    if e2 != e and n2 == n:
        n2 += align
    if e2 != e:
        pad = e2 - e
        iot = jnp.arange(pad, dtype=jnp.int32)
        pad_src = iot % jnp.int32(n)
        pad_dst = jnp.int32(n) + iot % jnp.int32(max(n2 - n, 1))
        ei = jnp.concatenate(
            [ei, jnp.stack([pad_src, pad_dst])], axis=1)
    src2d = ei[0].reshape(e2 // c, c)
    dst2d = ei[1].reshape(e2 // c, c)
    e = e2

    pre1 = _tc_pre(x, Wr1, bl1.reshape(1, d), block=block)
    p1, degp = _sc_aggregate(n2, n, d, e, c, True)(x, src2d, dst2d)
    h, invd = _tc_post1(p1, degp.reshape(NC, n2, 1), pre1, Wl1, block=block)
    pre2 = _tc_pre(h, Wr2, bl2.reshape(1, d), block=block)
    (p2,) = _sc_aggregate(n2, n, d, e, c, False)(h, src2d, dst2d)
    return _tc_post2(p2, invd, pre2, Wl2, block=block)


# speedup vs baseline: 1.1189x; 1.1189x over previous
import functools

import jax
import jax.numpy as jnp
from jax import lax
from jax.experimental import pallas as pl
from jax.experimental.pallas import tpu as pltpu
from jax.experimental.pallas import tpu_sc as plsc

NC = 2
NS = 16
NW = NC * NS
CHUNK = 64
NBUF = 4


@functools.lru_cache(maxsize=None)
def _sc_aggregate(n: int, nf: int, d: int, e: int, c: int, with_deg: bool):
    epw = e // NW
    assert epw * NW == e and epw % c == 0
    kpw = epw // c
    assert kpw % 8 == 0 and kpw % NBUF == 0
    rpt = n // NS
    assert rpt * NS == n and rpt % 8 == 0
    kg = next(g for g in range(min(kpw, 16), 0, -8)
              if kpw % g == 0 and g % NBUF == 0)
    zc = next(z for z in range(min(c, 128) - min(c, 128) % 8, 0, -8)
              if rpt % z == 0)
    nlane = d // 16
    caux = -(-c // 16) * 16

    mesh = plsc.VectorSubcoreMesh(core_axis_name="c", subcore_axis_name="s")
    out_type = [jax.ShapeDtypeStruct((NC, n, d), jnp.float32)]
    if with_deg:
        out_type.append(jax.ShapeDtypeStruct((NC, n), jnp.float32))

    scratch = (
        [pltpu.VMEM((kg, c), jnp.int32)] * 4
        + [pltpu.VMEM((c, d), jnp.float32)] * NBUF
        + [pltpu.VMEM_SHARED((n, d), jnp.float32)]
        + [pltpu.SemaphoreType.DMA] * (2 * NBUF + 1)
    )
    if with_deg:
        scratch += [
            pltpu.VMEM((caux,), jnp.float32),
            pltpu.VMEM_SHARED((n,), jnp.float32),
            pltpu.SemaphoreType.DMA,
        ]

    def body(feats_hbm, src_hbm, dst_hbm, *refs):
        if with_deg:
            (outp_hbm, outdeg_hbm, sv0, dv0, sv1, dv1, *rest) = refs
            aux1d, deg_sh, dsem = rest[-3:]
            rest = rest[:-3]
        else:
            (outp_hbm, sv0, dv0, sv1, dv1, *rest) = refs
        src_b = (sv0, sv1)
        dst_b = (dv0, dv1)
        rows = tuple(rest[:NBUF])
        agg_sh = rest[NBUF]
        gsem = tuple(rest[NBUF + 1:NBUF + 1 + NBUF])
        ssem = tuple(rest[NBUF + 1 + NBUF:NBUF + 1 + 2 * NBUF])
        stsem = rest[NBUF + 1 + 2 * NBUF]
        cid = lax.axis_index("c")
        sid = lax.axis_index("s")
        wid = cid * NS + sid

        zv = jnp.zeros((16,), jnp.float32)

        def zrow(r, _):
            for j in range(nlane):
                rows[0][r, pl.ds(j * 16, 16)] = zv
            return 0

        lax.fori_loop(0, c, zrow, 0)
        if with_deg:
            ov = jnp.ones((16,), jnp.float32)
            for j in range(caux // 16):
                aux1d[pl.ds(j * 16, 16)] = zv

        for t in range(rpt // zc):
            pltpu.sync_copy(rows[0].at[pl.ds(0, zc)],
                            agg_sh.at[pl.ds(sid * rpt + t * zc, zc)])
        if with_deg:
            for t in range(-(-rpt // caux)):
                w = min(caux, rpt - t * caux)
                pltpu.sync_copy(
                    aux1d.at[pl.ds(0, w)],
                    deg_sh.at[pl.ds(sid * rpt + t * caux, w)])
            for j in range(caux // 16):
                aux1d[pl.ds(j * 16, 16)] = ov
        plsc.subcore_barrier()

        def gather(sv, k, b):
            return pltpu.async_copy(feats_hbm.at[sv.at[k]], rows[b],
                                    gsem[b])

        def scatter(dv, k, b):
            pltpu.async_copy(rows[b], agg_sh.at[dv.at[k]], ssem[b],
                             add=True)
            if with_deg:
                pltpu.async_copy(aux1d.at[pl.ds(0, c)],
                                 deg_sh.at[dv.at[k]], dsem, add=True)

        def wait_g(b):
            pltpu.make_async_copy(feats_hbm.at[sv0.at[0]], rows[b],
                                  gsem[b]).wait()

        def wait_s(b):
            pltpu.make_async_copy(rows[b], agg_sh.at[dv0.at[0]],
                                  ssem[b]).wait()

        def stage(g, w):
            base = wid * kpw + g * kg
            pltpu.async_copy(src_hbm.at[pl.ds(base, kg)], src_b[w], stsem)
            pltpu.async_copy(dst_hbm.at[pl.ds(base, kg)], dst_b[w], stsem)

        def wait_stage():
            for _ in range(2):
                pltpu.make_async_copy(src_hbm.at[pl.ds(0, kg)], src_b[0],
                                      stsem).wait()

        nsteps = kg // NBUF
        ngroups = kpw // kg
        stage(0, 0)
        wait_stage()
        for b in range(NBUF):
            gather(sv0, b, b)
        for g in range(ngroups):
            sv, dv = src_b[g % 2], dst_b[g % 2]
            for b in range(NBUF):
                wait_g(b)
                scatter(dv, b, b)
            if g + 1 < ngroups:
                stage(g + 1, (g + 1) % 2)
            if nsteps > 1:
                for b in range(NBUF):
                    wait_s(b)
                    gather(sv, NBUF + b, b)

                def step(t, _):
                    k0 = NBUF * t
                    for b in range(NBUF):
                        wait_g(b)
                        scatter(dv, k0 + b, b)

                    @pl.when(t < nsteps - 1)
                    def _():
                        for b in range(NBUF):
                            wait_s(b)
                            gather(sv, k0 + NBUF + b, b)

                    return 0

                lax.fori_loop(1, nsteps, step, 0)
            if g + 1 < ngroups:
                wait_stage()
                for b in range(NBUF):
                    wait_s(b)
                    gather(src_b[(g + 1) % 2], b, b)
            else:
                for b in range(NBUF):
                    wait_s(b)
            if with_deg:
                def ddrain(t, _):
                    pltpu.make_async_copy(aux1d.at[pl.ds(0, c)],
                                          deg_sh.at[dv0.at[0]],
                                          dsem).wait()
                    return 0

                lax.fori_loop(0, kg, ddrain, 0)

        plsc.subcore_barrier()

        pltpu.sync_copy(agg_sh.at[pl.ds(sid * rpt, rpt)],
                        outp_hbm.at[cid, pl.ds(sid * rpt, rpt)])
        if with_deg:
            pltpu.sync_copy(deg_sh.at[pl.ds(sid * rpt, rpt)],
                            outdeg_hbm.at[cid, pl.ds(sid * rpt, rpt)])

    return pl.kernel(body, out_type=out_type, mesh=mesh,
                     scratch_types=scratch)


def _tc_pre(xin, Wr, bl, block: int):
    n, d = xin.shape

    def body(x_ref, wr_ref, bl_ref, o_ref):
        o_ref[...] = (jnp.dot(x_ref[...], wr_ref[...],
                              preferred_element_type=jnp.float32)
                      + bl_ref[...])

    return pl.pallas_call(
        body,
        grid=(n // block,),
        in_specs=[
            pl.BlockSpec((block, d), lambda i: (i, 0)),
            pl.BlockSpec((d, d), lambda i: (0, 0)),
            pl.BlockSpec((1, d), lambda i: (0, 0)),
        ],
        out_specs=pl.BlockSpec((block, d), lambda i: (i, 0)),
        out_shape=jax.ShapeDtypeStruct((n, d), jnp.float32),
    )(xin, Wr, bl)


def _tc_post1(p, degp, pre, Wl, block: int):
    n, d = pre.shape

    def body(p_ref, deg_ref, pre_ref, wl_ref, o_ref, inv_ref):
        dsum = deg_ref[0] + deg_ref[1]
        invd = 1.0 / jnp.maximum(dsum, 1.0)
        agg = (p_ref[0] + p_ref[1]) * invd
        y = (jnp.dot(agg, wl_ref[...], preferred_element_type=jnp.float32)
             + pre_ref[...])
        o_ref[...] = jnp.maximum(y, 0.0)
        inv_ref[...] = invd

    return pl.pallas_call(
        body,
        grid=(n // block,),
        in_specs=[
            pl.BlockSpec((2, block, d), lambda i: (0, i, 0)),
            pl.BlockSpec((2, block, 1), lambda i: (0, i, 0)),
            pl.BlockSpec((block, d), lambda i: (i, 0)),
            pl.BlockSpec((d, d), lambda i: (0, 0)),
        ],
        out_specs=[
            pl.BlockSpec((block, d), lambda i: (i, 0)),
            pl.BlockSpec((block, 1), lambda i: (i, 0)),
        ],
        out_shape=[
            jax.ShapeDtypeStruct((n, d), jnp.float32),
            jax.ShapeDtypeStruct((n, 1), jnp.float32),
        ],
    )(p, degp, pre, Wl)


def _tc_post2(p, invd, pre, Wl, block: int):
    n, d = pre.shape

    def body(p_ref, inv_ref, pre_ref, wl_ref, o_ref):
        agg = (p_ref[0] + p_ref[1]) * inv_ref[...]
        o_ref[...] = (
            jnp.dot(agg, wl_ref[...], preferred_element_type=jnp.float32)
            + pre_ref[...])

    return pl.pallas_call(
        body,
        grid=(n // block,),
        in_specs=[
            pl.BlockSpec((2, block, d), lambda i: (0, i, 0)),
            pl.BlockSpec((block, 1), lambda i: (i, 0)),
            pl.BlockSpec((block, d), lambda i: (i, 0)),
            pl.BlockSpec((d, d), lambda i: (0, 0)),
        ],
        out_specs=pl.BlockSpec((block, d), lambda i: (i, 0)),
        out_shape=jax.ShapeDtypeStruct((n, d), jnp.float32),
    )(p, invd, pre, Wl)


def kernel(x, edge_index, Wl1, bl1, Wr1, Wl2, bl2, Wr2):
    n, d = x.shape
    e = edge_index.shape[1]
    ei = edge_index.astype(jnp.int32)

    align = NS * 128
    n2 = -(-n // align) * align
    block = next(b for b in range(1024, 0, -8) if n % b == 0)

    c = CHUNK
    egrain = NW * c * 8
    e2 = -(-e // egrain) * egrain
    if e2 != e and n2 == n:
        n2 += align
    if e2 != e:
        pad = e2 - e
        iot = jnp.arange(pad, dtype=jnp.int32)
        pad_src = iot % jnp.int32(n)
        pad_dst = jnp.int32(n) + iot % jnp.int32(max(n2 - n, 1))
        ei = jnp.concatenate(
            [ei, jnp.stack([pad_src, pad_dst])], axis=1)
    src2d = ei[0].reshape(e2 // c, c)
    dst2d = ei[1].reshape(e2 // c, c)
    e = e2

    pre1 = _tc_pre(x, Wr1, bl1.reshape(1, d), block=block)
    p1, degp = _sc_aggregate(n2, n, d, e, c, True)(x, src2d, dst2d)
    h, invd = _tc_post1(p1, degp.reshape(NC, n2, 1), pre1, Wl1, block=block)
    pre2 = _tc_pre(h, Wr2, bl2.reshape(1, d), block=block)
    (p2,) = _sc_aggregate(n2, n, d, e, c, False)(h, src2d, dst2d)
    return _tc_post2(p2, invd, pre2, Wl2, block=block)


# speedup vs baseline: 1.1450x; 1.0233x over previous
import functools

import jax
import jax.numpy as jnp
from jax import lax
from jax.experimental import pallas as pl
from jax.experimental.pallas import tpu as pltpu
from jax.experimental.pallas import tpu_sc as plsc

NC = 2
NS = 16
NW = NC * NS
CHUNK = 80
NBUF = 4


@functools.lru_cache(maxsize=None)
def _sc_aggregate(n: int, nf: int, d: int, e: int, c: int, with_deg: bool):
    epw = e // NW
    assert epw * NW == e and epw % c == 0
    kpw = epw // c
    assert kpw % 8 == 0 and kpw % NBUF == 0
    rpt = n // NS
    assert rpt * NS == n and rpt % 8 == 0
    def _fits(g):
        per_tile = (NBUF * c * d + 4 * g * (-(-c // 128) * 128)
                    + (-(-c // 16) * 16))
        shared = n * d + (n if with_deg else 0)
        return NS * per_tile + shared <= 2_080_000

    kg = next(g for g in range(min(kpw, 16), 0, -8)
              if kpw % g == 0 and g % NBUF == 0 and _fits(g))
    zc = next(z for z in range(min(c, 128) - min(c, 128) % 8, 0, -8)
              if rpt % z == 0)
    nlane = d // 16
    caux = -(-c // 16) * 16

    mesh = plsc.VectorSubcoreMesh(core_axis_name="c", subcore_axis_name="s")
    out_type = [jax.ShapeDtypeStruct((NC, n, d), jnp.float32)]
    if with_deg:
        out_type.append(jax.ShapeDtypeStruct((NC, n), jnp.float32))

    scratch = (
        [pltpu.VMEM((kg, c), jnp.int32)] * 4
        + [pltpu.VMEM((c, d), jnp.float32)] * NBUF
        + [pltpu.VMEM_SHARED((n, d), jnp.float32)]
        + [pltpu.SemaphoreType.DMA] * (2 * NBUF + 1)
    )
    if with_deg:
        scratch += [
            pltpu.VMEM((caux,), jnp.float32),
            pltpu.VMEM_SHARED((n,), jnp.float32),
            pltpu.SemaphoreType.DMA,
        ]

    def body(feats_hbm, src_hbm, dst_hbm, *refs):
        if with_deg:
            (outp_hbm, outdeg_hbm, sv0, dv0, sv1, dv1, *rest) = refs
            aux1d, deg_sh, dsem = rest[-3:]
            rest = rest[:-3]
        else:
            (outp_hbm, sv0, dv0, sv1, dv1, *rest) = refs
        src_b = (sv0, sv1)
        dst_b = (dv0, dv1)
        rows = tuple(rest[:NBUF])
        agg_sh = rest[NBUF]
        gsem = tuple(rest[NBUF + 1:NBUF + 1 + NBUF])
        ssem = tuple(rest[NBUF + 1 + NBUF:NBUF + 1 + 2 * NBUF])
        stsem = rest[NBUF + 1 + 2 * NBUF]
        cid = lax.axis_index("c")
        sid = lax.axis_index("s")
        wid = cid * NS + sid

        zv = jnp.zeros((16,), jnp.float32)

        def zrow(r, _):
            for j in range(nlane):
                rows[0][r, pl.ds(j * 16, 16)] = zv
            return 0

        lax.fori_loop(0, c, zrow, 0)
        if with_deg:
            ov = jnp.ones((16,), jnp.float32)
            for j in range(caux // 16):
                aux1d[pl.ds(j * 16, 16)] = zv

        for t in range(rpt // zc):
            pltpu.sync_copy(rows[0].at[pl.ds(0, zc)],
                            agg_sh.at[pl.ds(sid * rpt + t * zc, zc)])
        if with_deg:
            for t in range(-(-rpt // caux)):
                w = min(caux, rpt - t * caux)
                pltpu.sync_copy(
                    aux1d.at[pl.ds(0, w)],
                    deg_sh.at[pl.ds(sid * rpt + t * caux, w)])
            for j in range(caux // 16):
                aux1d[pl.ds(j * 16, 16)] = ov
        plsc.subcore_barrier()

        def gather(sv, k, b):
            return pltpu.async_copy(feats_hbm.at[sv.at[k]], rows[b],
                                    gsem[b])

        def scatter(dv, k, b):
            pltpu.async_copy(rows[b], agg_sh.at[dv.at[k]], ssem[b],
                             add=True)
            if with_deg:
                pltpu.async_copy(aux1d.at[pl.ds(0, c)],
                                 deg_sh.at[dv.at[k]], dsem, add=True)

        def wait_g(b):
            pltpu.make_async_copy(feats_hbm.at[sv0.at[0]], rows[b],
                                  gsem[b]).wait()

        def wait_s(b):
            pltpu.make_async_copy(rows[b], agg_sh.at[dv0.at[0]],
                                  ssem[b]).wait()

        def stage(g, w):
            base = wid * kpw + g * kg
            pltpu.async_copy(src_hbm.at[pl.ds(base, kg)], src_b[w], stsem)
            pltpu.async_copy(dst_hbm.at[pl.ds(base, kg)], dst_b[w], stsem)

        def wait_stage():
            for _ in range(2):
                pltpu.make_async_copy(src_hbm.at[pl.ds(0, kg)], src_b[0],
                                      stsem).wait()

        nsteps = kg // NBUF
        ngroups = kpw // kg
        stage(0, 0)
        wait_stage()
        for b in range(NBUF):
            gather(sv0, b, b)
        for g in range(ngroups):
            sv, dv = src_b[g % 2], dst_b[g % 2]
            for b in range(NBUF):
                wait_g(b)
                scatter(dv, b, b)
            if g + 1 < ngroups:
                stage(g + 1, (g + 1) % 2)
            if nsteps > 1:
                for b in range(NBUF):
                    wait_s(b)
                    gather(sv, NBUF + b, b)

                def step(t, _):
                    k0 = NBUF * t
                    for b in range(NBUF):
                        wait_g(b)
                        scatter(dv, k0 + b, b)

                    @pl.when(t < nsteps - 1)
                    def _():
                        for b in range(NBUF):
                            wait_s(b)
                            gather(sv, k0 + NBUF + b, b)

                    return 0

                lax.fori_loop(1, nsteps, step, 0)
            if g + 1 < ngroups:
                wait_stage()
                for b in range(NBUF):
                    wait_s(b)
                    gather(src_b[(g + 1) % 2], b, b)
            else:
                for b in range(NBUF):
                    wait_s(b)
            if with_deg:
                def ddrain(t, _):
                    pltpu.make_async_copy(aux1d.at[pl.ds(0, c)],
                                          deg_sh.at[dv0.at[0]],
                                          dsem).wait()
                    return 0

                lax.fori_loop(0, kg, ddrain, 0)

        plsc.subcore_barrier()

        pltpu.sync_copy(agg_sh.at[pl.ds(sid * rpt, rpt)],
                        outp_hbm.at[cid, pl.ds(sid * rpt, rpt)])
        if with_deg:
            pltpu.sync_copy(deg_sh.at[pl.ds(sid * rpt, rpt)],
                            outdeg_hbm.at[cid, pl.ds(sid * rpt, rpt)])

    return pl.kernel(body, out_type=out_type, mesh=mesh,
                     scratch_types=scratch)


def _tc_pre(xin, Wr, bl, block: int):
    n, d = xin.shape

    def body(x_ref, wr_ref, bl_ref, o_ref):
        o_ref[...] = (jnp.dot(x_ref[...], wr_ref[...],
                              preferred_element_type=jnp.float32)
                      + bl_ref[...])

    return pl.pallas_call(
        body,
        grid=(n // block,),
        in_specs=[
            pl.BlockSpec((block, d), lambda i: (i, 0)),
            pl.BlockSpec((d, d), lambda i: (0, 0)),
            pl.BlockSpec((1, d), lambda i: (0, 0)),
        ],
        out_specs=pl.BlockSpec((block, d), lambda i: (i, 0)),
        out_shape=jax.ShapeDtypeStruct((n, d), jnp.float32),
    )(xin, Wr, bl)


def _tc_post1(p, degp, pre, Wl, block: int):
    n, d = pre.shape

    def body(p_ref, deg_ref, pre_ref, wl_ref, o_ref, inv_ref):
        dsum = deg_ref[0] + deg_ref[1]
        invd = 1.0 / jnp.maximum(dsum, 1.0)
        agg = (p_ref[0] + p_ref[1]) * invd
        y = (jnp.dot(agg, wl_ref[...], preferred_element_type=jnp.float32)
             + pre_ref[...])
        o_ref[...] = jnp.maximum(y, 0.0)
        inv_ref[...] = invd

    return pl.pallas_call(
        body,
        grid=(n // block,),
        in_specs=[
            pl.BlockSpec((2, block, d), lambda i: (0, i, 0)),
            pl.BlockSpec((2, block, 1), lambda i: (0, i, 0)),
            pl.BlockSpec((block, d), lambda i: (i, 0)),
            pl.BlockSpec((d, d), lambda i: (0, 0)),
        ],
        out_specs=[
            pl.BlockSpec((block, d), lambda i: (i, 0)),
            pl.BlockSpec((block, 1), lambda i: (i, 0)),
        ],
        out_shape=[
            jax.ShapeDtypeStruct((n, d), jnp.float32),
            jax.ShapeDtypeStruct((n, 1), jnp.float32),
        ],
    )(p, degp, pre, Wl)


def _tc_post2(p, invd, pre, Wl, block: int):
    n, d = pre.shape

    def body(p_ref, inv_ref, pre_ref, wl_ref, o_ref):
        agg = (p_ref[0] + p_ref[1]) * inv_ref[...]
        o_ref[...] = (
            jnp.dot(agg, wl_ref[...], preferred_element_type=jnp.float32)
            + pre_ref[...])

    return pl.pallas_call(
        body,
        grid=(n // block,),
        in_specs=[
            pl.BlockSpec((2, block, d), lambda i: (0, i, 0)),
            pl.BlockSpec((block, 1), lambda i: (i, 0)),
            pl.BlockSpec((block, d), lambda i: (i, 0)),
            pl.BlockSpec((d, d), lambda i: (0, 0)),
        ],
        out_specs=pl.BlockSpec((block, d), lambda i: (i, 0)),
        out_shape=jax.ShapeDtypeStruct((n, d), jnp.float32),
    )(p, invd, pre, Wl)


def kernel(x, edge_index, Wl1, bl1, Wr1, Wl2, bl2, Wr2):
    n, d = x.shape
    e = edge_index.shape[1]
    ei = edge_index.astype(jnp.int32)

    align = NS * 128
    n2 = -(-n // align) * align
    block = next(b for b in range(1024, 0, -8) if n % b == 0)

    c = CHUNK
    egrain = NW * c * 8
    e2 = -(-e // egrain) * egrain
    if e2 != e and n2 == n:
        n2 += align
    if e2 != e:
        pad = e2 - e
        iot = jnp.arange(pad, dtype=jnp.int32)
        pad_src = iot % jnp.int32(n)
        pad_dst = jnp.int32(n) + iot % jnp.int32(max(n2 - n, 1))
        ei = jnp.concatenate(
            [ei, jnp.stack([pad_src, pad_dst])], axis=1)
    src2d = ei[0].reshape(e2 // c, c)
    dst2d = ei[1].reshape(e2 // c, c)
    e = e2

    pre1 = _tc_pre(x, Wr1, bl1.reshape(1, d), block=block)
    p1, degp = _sc_aggregate(n2, n, d, e, c, True)(x, src2d, dst2d)
    h, invd = _tc_post1(p1, degp.reshape(NC, n2, 1), pre1, Wl1, block=block)
    pre2 = _tc_pre(h, Wr2, bl2.reshape(1, d), block=block)
    (p2,) = _sc_aggregate(n2, n, d, e, c, False)(h, src2d, dst2d)
    return _tc_post2(p2, invd, pre2, Wl2, block=block)
